# speedup vs baseline: 34.0038x; 34.0038x over previous
import jax
import jax.numpy as jnp
from jax.experimental import pallas as pl
from jax.experimental.pallas import tpu as pltpu

_B, _N = 64, 32768
_K1 = int(0.01 * (_N - 1)) + 1


def _mmcl_body(logits_ref, tgt_ref, out_ref, key_scratch):
    x = logits_ref[...]
    b = jax.lax.bitcast_convert_type(x, jnp.int32)
    key_scratch[...] = b ^ ((b >> 31) & jnp.int32(0x7FFFFFFF))

    def step(i, res):
        cand = res + (jnp.int32(1) << (jnp.int32(30) - i))
        cnt = jnp.sum((key_scratch[...] >= cand).astype(jnp.int32), axis=1,
                      keepdims=True)
        return jnp.where(cnt >= _K1, cand, res)

    res0 = jnp.full((_B, 1), jnp.int32(-2147483648))
    tcode = jax.lax.fori_loop(0, 31, step, res0)

    key = key_scratch[...]
    m = jnp.max(x, axis=1, keepdims=True)
    gt = key > tcode
    c = jnp.sum(gt.astype(jnp.int32), axis=1, keepdims=True)
    ex = jnp.where(gt, jnp.exp(10.0 * (x - m)), 0.0)
    big = jnp.sum(ex, axis=1, keepdims=True)

    tb = tcode ^ ((tcode >> 31) & jnp.int32(0x7FFFFFFF))
    t = jax.lax.bitcast_convert_type(tb, jnp.float32)

    cols = jax.lax.broadcasted_iota(jnp.int32, (_B, _N), 1)
    pos = jnp.sum(jnp.where(cols == tgt_ref[...], x, 0.0), axis=1,
                  keepdims=True)

    et = jnp.exp(10.0 * (t - m))
    ep = jnp.exp(10.0 * (pos - m))
    s = big + (_K1 - c).astype(jnp.float32) * et + jnp.where(
        pos < t, ep - et, 0.0)
    loss = jnp.log(s) + 10.0 * m - 10.0 * pos
    out_ref[0, 0] = jnp.sum(loss) / _B


@jax.jit
def kernel(logits, targets):
    tgt2 = targets.reshape(_B, 1).astype(jnp.int32)
    out = pl.pallas_call(
        _mmcl_body,
        out_shape=jax.ShapeDtypeStruct((1, 1), jnp.float32),
        in_specs=[
            pl.BlockSpec(memory_space=pltpu.VMEM),
            pl.BlockSpec(memory_space=pltpu.VMEM),
        ],
        out_specs=pl.BlockSpec(memory_space=pltpu.SMEM),
        scratch_shapes=[pltpu.VMEM((_B, _N), jnp.int32)],
    )(logits, tgt2)
    return out[0, 0]
